# initial kernel scaffold (unmeasured)
import jax
import jax.numpy as jnp
from jax import lax
from jax.experimental import pallas as pl
from jax.experimental.pallas import tpu as pltpu

Y = 4
T_SHARD = 2048
T_FULL = Y * T_SHARD
D = 1024
F = 2048
E_LOCAL = 4
BLK = 512
A_ROWS = T_SHARD // 128


def _neighbor_barrier(mx, mz, left, right):
    bsem = pltpu.get_barrier_semaphore()
    pl.semaphore_signal(bsem, inc=1, device_id=(mx, left, mz),
                        device_id_type=pl.DeviceIdType.MESH)
    pl.semaphore_signal(bsem, inc=1, device_id=(mx, right, mz),
                        device_id_type=pl.DeviceIdType.MESH)
    pl.semaphore_wait(bsem, 2)


def _ag_body(x_ref, a_ref, xf_ref, af_ref, sx, rx, sa, ra):
    my = lax.axis_index("y")
    mx = lax.axis_index("x")
    mz = lax.axis_index("z")
    left = lax.rem(my + Y - 1, Y)
    right = lax.rem(my + 1, Y)

    _neighbor_barrier(mx, mz, left, right)

    xf_ref[pl.ds(my * T_SHARD, T_SHARD), :] = x_ref[...]
    af_ref[pl.ds(my * A_ROWS, A_ROWS), :] = a_ref[...]

    for h in range(Y - 1):
        org = lax.rem(my + Y - h, Y)
        rdma_x = pltpu.make_async_remote_copy(
            src_ref=xf_ref.at[pl.ds(org * T_SHARD, T_SHARD), :],
            dst_ref=xf_ref.at[pl.ds(org * T_SHARD, T_SHARD), :],
            send_sem=sx.at[h],
            recv_sem=rx.at[h],
            device_id=(mx, right, mz),
            device_id_type=pl.DeviceIdType.MESH,
        )
        rdma_a = pltpu.make_async_remote_copy(
            src_ref=af_ref.at[pl.ds(org * A_ROWS, A_ROWS), :],
            dst_ref=af_ref.at[pl.ds(org * A_ROWS, A_ROWS), :],
            send_sem=sa.at[h],
            recv_sem=ra.at[h],
            device_id=(mx, right, mz),
            device_id_type=pl.DeviceIdType.MESH,
        )
        rdma_x.start()
        rdma_a.start()
        rdma_x.wait()
        rdma_a.wait()


def _moe_body(af_ref, xf_ref, w1_ref, w2_ref, out_ref):
    e = pl.program_id(1)
    my = lax.axis_index("y")
    a = af_ref[...].reshape(BLK)
    e_id = my * E_LOCAL + e
    xm = jnp.where((a == e_id)[:, None], xf_ref[...], 0.0)
    h = jnp.maximum(
        jnp.dot(xm, w1_ref[0], preferred_element_type=jnp.float32), 0.0
    )
    contrib = jnp.dot(h, w2_ref[0], preferred_element_type=jnp.float32)

    @pl.when(e == 0)
    def _():
        out_ref[...] = contrib

    @pl.when(e != 0)
    def _():
        out_ref[...] += contrib


def _rs_body(p_ref, out_ref, recv_ref, acc_ref, ss, rs):
    my = lax.axis_index("y")
    mx = lax.axis_index("x")
    mz = lax.axis_index("z")
    left = lax.rem(my + Y - 1, Y)
    right = lax.rem(my + 1, Y)

    _neighbor_barrier(mx, mz, left, right)

    org0 = left
    r0 = pltpu.make_async_remote_copy(
        src_ref=p_ref.at[pl.ds(org0 * T_SHARD, T_SHARD), :],
        dst_ref=recv_ref.at[0],
        send_sem=ss.at[0],
        recv_sem=rs.at[0],
        device_id=(mx, right, mz),
        device_id_type=pl.DeviceIdType.MESH,
    )
    r0.start()
    r0.wait()

    for s in range(1, Y - 1):
        org = lax.rem(my + Y - 1 - s, Y)
        acc_ref[...] = (
            p_ref[pl.ds(org * T_SHARD, T_SHARD), :] + recv_ref[s - 1]
        )
        r = pltpu.make_async_remote_copy(
            src_ref=acc_ref,
            dst_ref=recv_ref.at[s],
            send_sem=ss.at[s],
            recv_sem=rs.at[s],
            device_id=(mx, right, mz),
            device_id_type=pl.DeviceIdType.MESH,
        )
        r.start()
        r.wait()

    out_ref[...] = p_ref[pl.ds(my * T_SHARD, T_SHARD), :] + recv_ref[Y - 2]


def kernel(x, assign, W1, W2):
    a2 = assign.reshape(A_ROWS, 128)

    xf, af = pl.pallas_call(
        _ag_body,
        out_shape=(
            jax.ShapeDtypeStruct((T_FULL, D), jnp.float32),
            jax.ShapeDtypeStruct((Y * A_ROWS, 128), jnp.int32),
        ),
        in_specs=[
            pl.BlockSpec(memory_space=pltpu.VMEM),
            pl.BlockSpec(memory_space=pltpu.VMEM),
        ],
        out_specs=(
            pl.BlockSpec(memory_space=pltpu.VMEM),
            pl.BlockSpec(memory_space=pltpu.VMEM),
        ),
        scratch_shapes=[
            pltpu.SemaphoreType.DMA((Y - 1,)),
            pltpu.SemaphoreType.DMA((Y - 1,)),
            pltpu.SemaphoreType.DMA((Y - 1,)),
            pltpu.SemaphoreType.DMA((Y - 1,)),
        ],
        compiler_params=pltpu.CompilerParams(collective_id=0),
    )(x, a2)

    n_blk = T_FULL // BLK
    partial = pl.pallas_call(
        _moe_body,
        grid=(n_blk, E_LOCAL),
        in_specs=[
            pl.BlockSpec((BLK // 128, 128), lambda m, e: (m, 0)),
            pl.BlockSpec((BLK, D), lambda m, e: (m, 0)),
            pl.BlockSpec((1, D, F), lambda m, e: (e, 0, 0)),
            pl.BlockSpec((1, F, D), lambda m, e: (e, 0, 0)),
        ],
        out_specs=pl.BlockSpec((BLK, D), lambda m, e: (m, 0)),
        out_shape=jax.ShapeDtypeStruct((T_FULL, D), jnp.float32),
    )(af, xf, W1, W2)

    out = pl.pallas_call(
        _rs_body,
        out_shape=jax.ShapeDtypeStruct((T_SHARD, D), jnp.float32),
        in_specs=[pl.BlockSpec(memory_space=pltpu.VMEM)],
        out_specs=pl.BlockSpec(memory_space=pltpu.VMEM),
        scratch_shapes=[
            pltpu.VMEM((Y - 1, T_SHARD, D), jnp.float32),
            pltpu.VMEM((T_SHARD, D), jnp.float32),
            pltpu.SemaphoreType.DMA((Y - 1,)),
            pltpu.SemaphoreType.DMA((Y - 1,)),
        ],
        compiler_params=pltpu.CompilerParams(collective_id=1),
    )(partial)

    return out


# baseline (device time: 939939 ns/iter reference)
import jax
import jax.numpy as jnp
from jax import lax
from jax.experimental import pallas as pl
from jax.experimental.pallas import tpu as pltpu

Y = 4
T_SHARD = 2048
T_FULL = Y * T_SHARD
D = 1024
F = 2048
E_LOCAL = 4
BLK = 1024


def _neighbor_barrier(mx, mz, left, right):
    bsem = pltpu.get_barrier_semaphore()
    pl.semaphore_signal(bsem, inc=1, device_id=(mx, left, mz),
                        device_id_type=pl.DeviceIdType.MESH)
    pl.semaphore_signal(bsem, inc=1, device_id=(mx, right, mz),
                        device_id_type=pl.DeviceIdType.MESH)
    pl.semaphore_wait(bsem, 2)


def _ag_body(x_ref, a_ref, xf_ref, af_ref, sx, rx, sa, ra):
    my = lax.axis_index("y")
    mx = lax.axis_index("x")
    mz = lax.axis_index("z")
    left = lax.rem(my + Y - 1, Y)
    right = lax.rem(my + 1, Y)

    _neighbor_barrier(mx, mz, left, right)

    xf_ref[pl.ds(my * T_SHARD, T_SHARD), :] = x_ref[...]
    af_ref[pl.ds(my * T_SHARD, T_SHARD), :] = a_ref[...]

    for h in range(Y - 1):
        org = lax.rem(my + Y - h, Y)
        rdma_x = pltpu.make_async_remote_copy(
            src_ref=xf_ref.at[pl.ds(org * T_SHARD, T_SHARD), :],
            dst_ref=xf_ref.at[pl.ds(org * T_SHARD, T_SHARD), :],
            send_sem=sx.at[h],
            recv_sem=rx.at[h],
            device_id=(mx, right, mz),
            device_id_type=pl.DeviceIdType.MESH,
        )
        rdma_a = pltpu.make_async_remote_copy(
            src_ref=af_ref.at[pl.ds(org * T_SHARD, T_SHARD), :],
            dst_ref=af_ref.at[pl.ds(org * T_SHARD, T_SHARD), :],
            send_sem=sa.at[h],
            recv_sem=ra.at[h],
            device_id=(mx, right, mz),
            device_id_type=pl.DeviceIdType.MESH,
        )
        rdma_x.start()
        rdma_a.start()
        rdma_x.wait()
        rdma_a.wait()


def _moe_body(af_ref, xf_ref, w1_ref, w2_ref, out_ref):
    e = pl.program_id(1)
    my = lax.axis_index("y")
    e_id = my * E_LOCAL + e
    mask = af_ref[:, 0:1] == e_id
    xm = jnp.where(mask, xf_ref[...], 0.0)
    h = jnp.maximum(
        jnp.dot(xm, w1_ref[0], preferred_element_type=jnp.float32), 0.0
    )
    contrib = jnp.dot(h, w2_ref[0], preferred_element_type=jnp.float32)

    @pl.when(e == 0)
    def _():
        out_ref[...] = contrib

    @pl.when(e != 0)
    def _():
        out_ref[...] += contrib


def _rs_body(p_ref, out_ref, chunk_ref, recv_ref, ss, rs, lsem):
    my = lax.axis_index("y")
    mx = lax.axis_index("x")
    mz = lax.axis_index("z")
    left = lax.rem(my + Y - 1, Y)
    right = lax.rem(my + 1, Y)

    _neighbor_barrier(mx, mz, left, right)

    org0 = left
    r0 = pltpu.make_async_remote_copy(
        src_ref=p_ref.at[pl.ds(org0 * T_SHARD, T_SHARD), :],
        dst_ref=recv_ref.at[0],
        send_sem=ss.at[0],
        recv_sem=rs.at[0],
        device_id=(mx, right, mz),
        device_id_type=pl.DeviceIdType.MESH,
    )
    r0.start()
    r0.wait()

    for s in range(1, Y - 1):
        org = lax.rem(my + Y - 1 - s, Y)
        cp = pltpu.make_async_copy(
            p_ref.at[pl.ds(org * T_SHARD, T_SHARD), :], chunk_ref, lsem
        )
        cp.start()
        cp.wait()
        chunk_ref[...] += recv_ref[s - 1]
        r = pltpu.make_async_remote_copy(
            src_ref=chunk_ref,
            dst_ref=recv_ref.at[s],
            send_sem=ss.at[s],
            recv_sem=rs.at[s],
            device_id=(mx, right, mz),
            device_id_type=pl.DeviceIdType.MESH,
        )
        r.start()
        r.wait()

    cp = pltpu.make_async_copy(
        p_ref.at[pl.ds(my * T_SHARD, T_SHARD), :], chunk_ref, lsem
    )
    cp.start()
    cp.wait()
    out_ref[...] = chunk_ref[...] + recv_ref[Y - 2]


def kernel(x, assign, W1, W2):
    a2 = jnp.broadcast_to(assign[:, None], (T_SHARD, 128)).astype(jnp.int32)

    xf, af = pl.pallas_call(
        _ag_body,
        out_shape=(
            jax.ShapeDtypeStruct((T_FULL, D), jnp.float32),
            jax.ShapeDtypeStruct((T_FULL, 128), jnp.int32),
        ),
        in_specs=[
            pl.BlockSpec(memory_space=pltpu.VMEM),
            pl.BlockSpec(memory_space=pltpu.VMEM),
        ],
        out_specs=(
            pl.BlockSpec(memory_space=pltpu.VMEM),
            pl.BlockSpec(memory_space=pltpu.VMEM),
        ),
        scratch_shapes=[
            pltpu.SemaphoreType.DMA((Y - 1,)),
            pltpu.SemaphoreType.DMA((Y - 1,)),
            pltpu.SemaphoreType.DMA((Y - 1,)),
            pltpu.SemaphoreType.DMA((Y - 1,)),
        ],
        compiler_params=pltpu.CompilerParams(
            collective_id=0, vmem_limit_bytes=100 * 1024 * 1024
        ),
    )(x, a2)

    n_blk = T_FULL // BLK
    partial = pl.pallas_call(
        _moe_body,
        grid=(n_blk, E_LOCAL),
        in_specs=[
            pl.BlockSpec((BLK, 128), lambda m, e: (m, 0)),
            pl.BlockSpec((BLK, D), lambda m, e: (m, 0)),
            pl.BlockSpec((1, D, F), lambda m, e: (e, 0, 0)),
            pl.BlockSpec((1, F, D), lambda m, e: (e, 0, 0)),
        ],
        out_specs=pl.BlockSpec((BLK, D), lambda m, e: (m, 0)),
        out_shape=jax.ShapeDtypeStruct((T_FULL, D), jnp.float32),
        compiler_params=pltpu.CompilerParams(
            vmem_limit_bytes=100 * 1024 * 1024
        ),
    )(af, xf, W1, W2)

    out = pl.pallas_call(
        _rs_body,
        out_shape=jax.ShapeDtypeStruct((T_SHARD, D), jnp.float32),
        in_specs=[pl.BlockSpec(memory_space=pl.ANY)],
        out_specs=pl.BlockSpec(memory_space=pltpu.VMEM),
        scratch_shapes=[
            pltpu.VMEM((T_SHARD, D), jnp.float32),
            pltpu.VMEM((Y - 1, T_SHARD, D), jnp.float32),
            pltpu.SemaphoreType.DMA((Y - 1,)),
            pltpu.SemaphoreType.DMA((Y - 1,)),
            pltpu.SemaphoreType.DMA,
        ],
        compiler_params=pltpu.CompilerParams(
            collective_id=1, vmem_limit_bytes=100 * 1024 * 1024
        ),
    )(partial)

    return out
